# baseline (speedup 1.0000x reference)
import dataclasses

import jax
import jax.numpy as jnp
from jax.experimental import pallas as pl
from jax.experimental.pallas import tpu as pltpu
from jax.experimental.pallas import tpu_sc as plsc

K = 8192
D = 32
N = 9216
T = 256
NSUB = 32
TOK = N // NSUB
BETA = 0.25


def _sc_compiler_params():
    cp = pltpu.CompilerParams()
    if "needs_layout_passes" in pltpu.CompilerParams.__dataclass_fields__:
        cp = dataclasses.replace(cp, needs_layout_passes=False)
    return cp


def _argmin_onehot_body(x_ref, x2_ref, c2_ref, cbt_ref, oh_ref, idx_ref):
    if True:
        iota0 = jax.lax.broadcasted_iota(jnp.int32, (T, K), 1)
        oh_ref[...] = (iota0 == x_ref[0, 0].astype(jnp.int32)).astype(jnp.float32)
        idx_ref[...] = jnp.zeros((T, 1), jnp.int32)
        return
    xf = x_ref[...]
    xb2 = (2.0 * xf).astype(jnp.bfloat16)
    cross2 = jax.lax.dot_general(
        xb2, cbt_ref[...], (((1,), (0,)), ((), ())),
        preferred_element_type=jnp.float32)
    dist = (x2_ref[...] + c2_ref[...]) - cross2

    m = jnp.min(dist, axis=1, keepdims=True)
    iota = jax.lax.broadcasted_iota(jnp.int32, (T, K), 1)
    idxv = jnp.min(jnp.where(dist == m, iota.astype(jnp.float32), float(K)),
                   axis=1, keepdims=True)
    idxi = idxv.astype(jnp.int32)
    oh_ref[...] = (iota == idxi).astype(jnp.float32)
    idx_ref[...] = idxi


def _sc_lookup_kernel(idx_hbm, x_hbm, cb_hbm, st_hbm, hist_hbm, loss_hbm,
                      idx_vmem, x_vmem, xq_vmem, st_vmem, hist_vmem, acc_vmem):
    c = jax.lax.axis_index("c")
    s = jax.lax.axis_index("s")
    base = (c * 16 + s) * TOK

    pltpu.sync_copy(idx_hbm.at[pl.ds(base, TOK)], idx_vmem)
    pltpu.sync_copy(x_hbm.at[pl.ds(base, TOK)], x_vmem)
    pltpu.sync_copy(cb_hbm.at[idx_vmem], xq_vmem)

    @pl.loop(0, K, step=16)
    def _zero(j):
        hist_vmem[pl.ds(j, 16)] = jnp.zeros((16,), jnp.float32)

    acc_vmem[...] = jnp.zeros((16,), jnp.float32)

    @pl.loop(0, TOK, step=16)
    def _hist(t):
        plsc.addupdate_scatter(hist_vmem, [idx_vmem[pl.ds(t, 16)]],
                               jnp.ones((16,), jnp.float32))

    @pl.loop(0, TOK)
    def _rows(r):
        @pl.loop(0, D, step=16)
        def _cols(cc):
            xv = x_vmem[r, pl.ds(cc, 16)]
            qv = xq_vmem[r, pl.ds(cc, 16)]
            dv = qv - xv
            st_vmem[r, pl.ds(cc, 16)] = xv + dv
            acc_vmem[...] += dv * dv

    pltpu.sync_copy(st_vmem, st_hbm.at[pl.ds(base, TOK)])
    pltpu.sync_copy(hist_vmem, hist_hbm.at[c * 16 + s])
    pltpu.sync_copy(acc_vmem, loss_hbm.at[c * 16 + s])


def _finish_body(hp_ref, lp_ref, vq_ref, cl_ref, cm_ref, perp_ref):
    hist = jnp.sum(hp_ref[...], axis=0, keepdims=True)
    p = hist / N
    ent = jnp.sum(p * jnp.log(p + 1e-10), axis=(0, 1), keepdims=True)
    perp_ref[...] = jnp.exp(-ent)
    loss = jnp.sum(lp_ref[...], axis=(0, 1), keepdims=True)
    mse = loss / (N * D)
    cl_ref[...] = mse
    cm_ref[...] = mse
    vq_ref[...] = mse + mse * BETA


def kernel(x, codebook):
    b, d, h, w = x.shape
    xt = jnp.transpose(x, (0, 2, 3, 1))
    x_flat = xt.reshape(-1, d)
    x2 = jnp.sum(x_flat ** 2, axis=1, keepdims=True)
    c2 = jnp.sum(codebook ** 2, axis=1).reshape(1, K)
    cbt_bf = codebook.astype(jnp.bfloat16).T

    one_hot, idx2d = pl.pallas_call(
        _argmin_onehot_body,
        grid=(N // T,),
        in_specs=[
            pl.BlockSpec((T, D), lambda i: (i, 0)),
            pl.BlockSpec((T, 1), lambda i: (i, 0)),
            pl.BlockSpec((1, K), lambda i: (0, 0)),
            pl.BlockSpec((D, K), lambda i: (0, 0)),
        ],
        out_specs=[
            pl.BlockSpec((T, K), lambda i: (i, 0)),
            pl.BlockSpec((T, 1), lambda i: (i, 0)),
        ],
        out_shape=[
            jax.ShapeDtypeStruct((N, K), jnp.float32),
            jax.ShapeDtypeStruct((N, 1), jnp.int32),
        ],
        compiler_params=pltpu.CompilerParams(
            dimension_semantics=("parallel",)),
    )(x_flat, x2, c2, cbt_bf)

    idx = idx2d.reshape(N)

    sc_kernel = pl.kernel(
        _sc_lookup_kernel,
        out_type=[
            jax.ShapeDtypeStruct((N, D), jnp.float32),
            jax.ShapeDtypeStruct((NSUB, K), jnp.float32),
            jax.ShapeDtypeStruct((NSUB, 16), jnp.float32),
        ],
        mesh=plsc.VectorSubcoreMesh(core_axis_name="c", subcore_axis_name="s"),
        compiler_params=_sc_compiler_params(),
        scratch_types=[
            pltpu.VMEM((TOK,), jnp.int32),
            pltpu.VMEM((TOK, D), jnp.float32),
            pltpu.VMEM((TOK, 128), jnp.float32),
            pltpu.VMEM((TOK, D), jnp.float32),
            pltpu.VMEM((K,), jnp.float32),
            pltpu.VMEM((16,), jnp.float32),
        ],
    )
    cb_pad = jnp.pad(codebook, ((0, 0), (0, 128 - D)))
    st_flat, hist_parts, loss_parts = sc_kernel(idx, x_flat, cb_pad)

    vq, cl, cm, perp = pl.pallas_call(
        _finish_body,
        out_shape=[jax.ShapeDtypeStruct((1, 1), jnp.float32)] * 4,
    )(hist_parts, loss_parts)

    x_q_st = jnp.transpose(st_flat.reshape(b, h, w, d), (0, 3, 1, 2))
    return (vq.reshape(()), cl.reshape(()), cm.reshape(()),
            x_q_st, perp.reshape(()), one_hot)


# speedup vs baseline: 2.4469x; 2.4469x over previous
import dataclasses

import jax
import jax.numpy as jnp
from jax.experimental import pallas as pl
from jax.experimental.pallas import tpu as pltpu
from jax.experimental.pallas import tpu_sc as plsc

K = 8192
D = 32
N = 9216
T = 256
NSUB = 32
TOK = N // NSUB
BETA = 0.25


def _sc_compiler_params():
    cp = pltpu.CompilerParams()
    if "needs_layout_passes" in pltpu.CompilerParams.__dataclass_fields__:
        cp = dataclasses.replace(cp, needs_layout_passes=False)
    return cp


def _argmin_onehot_body(x_ref, x2_ref, c2_ref, cb_ref, iotaf_ref,
                        oh_ref, idx_ref):
    xf = x_ref[...]
    xb2 = (2.0 * xf).astype(jnp.bfloat16)
    cross2 = jax.lax.dot_general(
        xb2, cb_ref[...].astype(jnp.bfloat16), (((1,), (1,)), ((), ())),
        preferred_element_type=jnp.float32)
    dist = (x2_ref[...] + c2_ref[...]) - cross2

    m = jnp.min(dist, axis=1, keepdims=True)
    iota_f = iotaf_ref[...]
    idxv = jnp.min(jnp.where(dist == m, iota_f, float(K)),
                   axis=1, keepdims=True)
    oh_ref[...] = (iota_f == idxv).astype(jnp.float32)
    idx_ref[...] = idxv.astype(jnp.int32)


def _sc_lookup_kernel(idx_hbm, x_hbm, cb4_hbm, zero_hbm,
                      st_hbm, hist_hbm, loss_hbm,
                      idx_vmem, idx4_vmem, x_vmem, xq_vmem, st_vmem,
                      hist_vmem, acc_vmem):
    c = jax.lax.axis_index("c")
    s = jax.lax.axis_index("s")
    base = (c * 16 + s) * TOK

    pltpu.sync_copy(idx_hbm.at[pl.ds(base, TOK)], idx_vmem)
    pltpu.sync_copy(x_hbm.at[pl.ds(base, TOK)], x_vmem)
    pltpu.sync_copy(zero_hbm, hist_vmem)

    @pl.loop(0, TOK, step=16)
    def _i4(t):
        idx4_vmem[pl.ds(t, 16)] = idx_vmem[pl.ds(t, 16)] >> 2

    pltpu.sync_copy(cb4_hbm.at[idx4_vmem], xq_vmem)

    acc_vmem[...] = jnp.zeros((16,), jnp.float32)

    @pl.loop(0, TOK, step=16)
    def _hist(t):
        plsc.addupdate_scatter(hist_vmem, [idx_vmem[pl.ds(t, 16)]],
                               jnp.ones((16,), jnp.float32))

    iota16 = jax.lax.iota(jnp.int32, 16)

    @pl.loop(0, TOK, step=16)
    def _rows(t):
        rvec = t + iota16
        qv = (idx_vmem[pl.ds(t, 16)] & 3) * D

        @pl.loop(0, D)
        def _cols(c0):
            cvec = jnp.zeros((16,), jnp.int32) + c0
            xc = plsc.load_gather(x_vmem, [rvec, cvec])
            qc = plsc.load_gather(xq_vmem, [rvec, qv + cvec])
            dv = qc - xc
            plsc.store_scatter(st_vmem, [rvec, cvec], xc + dv)
            acc_vmem[...] += dv * dv

    pltpu.sync_copy(st_vmem, st_hbm.at[pl.ds(base, TOK)])
    pltpu.sync_copy(hist_vmem, hist_hbm.at[c * 16 + s])
    pltpu.sync_copy(acc_vmem, loss_hbm.at[c * 16 + s])


def _finish_body(hp_ref, lp_ref, vq_ref, cl_ref, cm_ref, perp_ref):
    hist = jnp.sum(hp_ref[...], axis=0, keepdims=True)
    p = hist / N
    ent = jnp.sum(p * jnp.log(p + 1e-10), axis=(0, 1), keepdims=True)
    perp_ref[...] = jnp.exp(-ent)
    loss = jnp.sum(lp_ref[...], axis=(0, 1), keepdims=True)
    mse = loss / (N * D)
    cl_ref[...] = mse
    cm_ref[...] = mse
    vq_ref[...] = mse + mse * BETA


def kernel(x, codebook):
    b, d, h, w = x.shape
    xt = jnp.transpose(x, (0, 2, 3, 1))
    x_flat = xt.reshape(-1, d)
    x2 = jnp.sum(x_flat ** 2, axis=1, keepdims=True)
    c2 = jnp.sum(codebook ** 2, axis=1).reshape(1, K)
    iota_f = jnp.arange(K, dtype=jnp.float32).reshape(1, K)

    one_hot, idx2d = pl.pallas_call(
        _argmin_onehot_body,
        grid=(N // T,),
        in_specs=[
            pl.BlockSpec((T, D), lambda i: (i, 0)),
            pl.BlockSpec((T, 1), lambda i: (i, 0)),
            pl.BlockSpec((1, K), lambda i: (0, 0)),
            pl.BlockSpec((K, D), lambda i: (0, 0)),
            pl.BlockSpec((1, K), lambda i: (0, 0)),
        ],
        out_specs=[
            pl.BlockSpec((T, K), lambda i: (i, 0)),
            pl.BlockSpec((T, 1), lambda i: (i, 0)),
        ],
        out_shape=[
            jax.ShapeDtypeStruct((N, K), jnp.float32),
            jax.ShapeDtypeStruct((N, 1), jnp.int32),
        ],
        compiler_params=pltpu.CompilerParams(
            dimension_semantics=("parallel",)),
    )(x_flat, x2, c2, codebook, iota_f)

    idx = idx2d.reshape(N)

    sc_kernel = pl.kernel(
        _sc_lookup_kernel,
        out_type=[
            jax.ShapeDtypeStruct((N, D), jnp.float32),
            jax.ShapeDtypeStruct((NSUB, K), jnp.float32),
            jax.ShapeDtypeStruct((NSUB, 16), jnp.float32),
        ],
        mesh=plsc.VectorSubcoreMesh(core_axis_name="c", subcore_axis_name="s"),
        compiler_params=_sc_compiler_params(),
        scratch_types=[
            pltpu.VMEM((TOK,), jnp.int32),
            pltpu.VMEM((TOK,), jnp.int32),
            pltpu.VMEM((TOK, D), jnp.float32),
            pltpu.VMEM((TOK, 128), jnp.float32),
            pltpu.VMEM((TOK, D), jnp.float32),
            pltpu.VMEM((K,), jnp.float32),
            pltpu.VMEM((16,), jnp.float32),
        ],
    )
    cb4 = codebook.reshape(K // 4, 4 * D)
    zero_k = jnp.zeros((K,), jnp.float32)
    st_flat, hist_parts, loss_parts = sc_kernel(idx, x_flat, cb4, zero_k)

    vq, cl, cm, perp = pl.pallas_call(
        _finish_body,
        out_shape=[jax.ShapeDtypeStruct((1, 1), jnp.float32)] * 4,
    )(hist_parts, loss_parts)

    x_q_st = jnp.transpose(st_flat.reshape(b, h, w, d), (0, 3, 1, 2))
    return (vq.reshape(()), cl.reshape(()), cm.reshape(()),
            x_q_st, perp.reshape(()), one_hot)


# speedup vs baseline: 2.6621x; 1.0880x over previous
import dataclasses

import jax
import jax.numpy as jnp
from jax.experimental import pallas as pl
from jax.experimental.pallas import tpu as pltpu
from jax.experimental.pallas import tpu_sc as plsc

K = 8192
D = 32
N = 9216
T = 256
NSUB = 32
TOK = N // NSUB
BETA = 0.25


def _sc_compiler_params():
    cp = pltpu.CompilerParams()
    if "needs_layout_passes" in pltpu.CompilerParams.__dataclass_fields__:
        cp = dataclasses.replace(cp, needs_layout_passes=False)
    return cp


def _argmin_onehot_body(x_ref, x2_ref, c2_ref, cbt_ref, oh_ref, idx_ref):
    xf = x_ref[...]
    xb2 = (2.0 * xf).astype(jnp.bfloat16)
    cross2 = jax.lax.dot_general(
        xb2, cbt_ref[...], (((1,), (0,)), ((), ())),
        preferred_element_type=jnp.float32)
    dist = (x2_ref[...] + c2_ref[...]) - cross2

    m = jnp.min(dist, axis=1, keepdims=True)
    iota = jax.lax.broadcasted_iota(jnp.int32, (T, K), 1)
    idxv = jnp.min(jnp.where(dist == m, iota.astype(jnp.float32), float(K)),
                   axis=1, keepdims=True)
    idxi = idxv.astype(jnp.int32)
    oh_ref[...] = (iota == idxi).astype(jnp.float32)
    idx_ref[...] = idxi


def _sc_lookup_kernel(idx_hbm, x_hbm, cb_hbm, zero_hbm,
                      st_hbm, hist_hbm, loss_hbm,
                      idx_vmem, x_vmem, xq_vmem, st_vmem, hist_vmem, acc_vmem):
    c = jax.lax.axis_index("c")
    s = jax.lax.axis_index("s")
    base = (c * 16 + s) * TOK

    pltpu.sync_copy(idx_hbm.at[pl.ds(base, TOK)], idx_vmem)
    pltpu.sync_copy(x_hbm.at[pl.ds(base, TOK)], x_vmem)
    pltpu.sync_copy(zero_hbm, hist_vmem)
    pltpu.sync_copy(cb_hbm.at[idx_vmem], xq_vmem)

    acc_vmem[...] = jnp.zeros((16,), jnp.float32)

    @pl.loop(0, TOK, step=16)
    def _hist(t):
        plsc.addupdate_scatter(hist_vmem, [idx_vmem[pl.ds(t, 16)]],
                               jnp.ones((16,), jnp.float32))

    @pl.loop(0, TOK)
    def _rows(r):
        @pl.loop(0, D, step=16)
        def _cols(cc):
            xv = x_vmem[r, pl.ds(cc, 16)]
            qv = xq_vmem[r, pl.ds(cc, 16)]
            dv = qv - xv
            st_vmem[r, pl.ds(cc, 16)] = xv + dv
            acc_vmem[...] += dv * dv

    pltpu.sync_copy(st_vmem, st_hbm.at[pl.ds(base, TOK)])
    pltpu.sync_copy(hist_vmem, hist_hbm.at[c * 16 + s])
    pltpu.sync_copy(acc_vmem, loss_hbm.at[c * 16 + s])


def _finish_body(hp_ref, lp_ref, vq_ref, cl_ref, cm_ref, perp_ref):
    hist = jnp.sum(hp_ref[...], axis=0, keepdims=True)
    p = hist / N
    ent = jnp.sum(p * jnp.log(p + 1e-10), axis=(0, 1), keepdims=True)
    perp_ref[...] = jnp.exp(-ent)
    loss = jnp.sum(lp_ref[...], axis=(0, 1), keepdims=True)
    mse = loss / (N * D)
    cl_ref[...] = mse
    cm_ref[...] = mse
    vq_ref[...] = mse + mse * BETA


def kernel(x, codebook):
    b, d, h, w = x.shape
    xt = jnp.transpose(x, (0, 2, 3, 1))
    x_flat = xt.reshape(-1, d)
    x2 = jnp.sum(x_flat ** 2, axis=1, keepdims=True)
    c2 = jnp.sum(codebook ** 2, axis=1).reshape(1, K)
    cbt_bf = codebook.astype(jnp.bfloat16).T

    one_hot, idx2d = pl.pallas_call(
        _argmin_onehot_body,
        grid=(N // T,),
        in_specs=[
            pl.BlockSpec((T, D), lambda i: (i, 0)),
            pl.BlockSpec((T, 1), lambda i: (i, 0)),
            pl.BlockSpec((1, K), lambda i: (0, 0)),
            pl.BlockSpec((D, K), lambda i: (0, 0)),
        ],
        out_specs=[
            pl.BlockSpec((T, K), lambda i: (i, 0)),
            pl.BlockSpec((T, 1), lambda i: (i, 0)),
        ],
        out_shape=[
            jax.ShapeDtypeStruct((N, K), jnp.float32),
            jax.ShapeDtypeStruct((N, 1), jnp.int32),
        ],
        compiler_params=pltpu.CompilerParams(
            dimension_semantics=("parallel",)),
    )(x_flat, x2, c2, cbt_bf)

    idx = idx2d.reshape(N)

    sc_kernel = pl.kernel(
        _sc_lookup_kernel,
        out_type=[
            jax.ShapeDtypeStruct((N, D), jnp.float32),
            jax.ShapeDtypeStruct((NSUB, K), jnp.float32),
            jax.ShapeDtypeStruct((NSUB, 16), jnp.float32),
        ],
        mesh=plsc.VectorSubcoreMesh(core_axis_name="c", subcore_axis_name="s"),
        compiler_params=_sc_compiler_params(),
        scratch_types=[
            pltpu.VMEM((TOK,), jnp.int32),
            pltpu.VMEM((TOK, D), jnp.float32),
            pltpu.VMEM((TOK, 128), jnp.float32),
            pltpu.VMEM((TOK, D), jnp.float32),
            pltpu.VMEM((K,), jnp.float32),
            pltpu.VMEM((16,), jnp.float32),
        ],
    )
    cb_pad = jnp.pad(codebook, ((0, 0), (0, 128 - D)))
    zero_k = jnp.zeros((K,), jnp.float32)
    st_flat, hist_parts, loss_parts = sc_kernel(idx, x_flat, cb_pad, zero_k)

    vq, cl, cm, perp = pl.pallas_call(
        _finish_body,
        out_shape=[jax.ShapeDtypeStruct((1, 1), jnp.float32)] * 4,
    )(hist_parts, loss_parts)

    x_q_st = jnp.transpose(st_flat.reshape(b, h, w, d), (0, 3, 1, 2))
    return (vq.reshape(()), cl.reshape(()), cm.reshape(()),
            x_q_st, perp.reshape(()), one_hot)


# speedup vs baseline: 2.6917x; 1.0111x over previous
import dataclasses

import jax
import jax.numpy as jnp
from jax.experimental import pallas as pl
from jax.experimental.pallas import tpu as pltpu
from jax.experimental.pallas import tpu_sc as plsc

K = 8192
D = 32
N = 9216
T = 288
NSUB = 32
TOK = N // NSUB
BETA = 0.25


def _sc_compiler_params():
    cp = pltpu.CompilerParams()
    if "needs_layout_passes" in pltpu.CompilerParams.__dataclass_fields__:
        cp = dataclasses.replace(cp, needs_layout_passes=False)
    return cp


def _argmin_onehot_body(x_ref, x2_ref, c2_ref, cbt_ref, oh_ref, idx_ref):
    xf = x_ref[...]
    xb2 = (2.0 * xf).astype(jnp.bfloat16)
    cross2 = jax.lax.dot_general(
        xb2, cbt_ref[...], (((1,), (0,)), ((), ())),
        preferred_element_type=jnp.float32)
    dist = (x2_ref[...] + c2_ref[...]) - cross2

    m = jnp.min(dist, axis=1, keepdims=True)
    iota = jax.lax.broadcasted_iota(jnp.int32, (T, K), 1)
    idxv = jnp.min(jnp.where(dist == m, iota.astype(jnp.float32), float(K)),
                   axis=1, keepdims=True)
    idxi = idxv.astype(jnp.int32)
    oh_ref[...] = (iota == idxi).astype(jnp.float32)
    idx_ref[...] = idxi


def _sc_lookup_kernel(idx_hbm, x_hbm, cb_hbm, zero_hbm,
                      st_hbm, hist_hbm, loss_hbm,
                      idx_vmem, x_vmem, xq_vmem, st_vmem, hist_vmem, acc_vmem,
                      sem_g, sem_x, sem_z):
    c = jax.lax.axis_index("c")
    s = jax.lax.axis_index("s")
    base = (c * 16 + s) * TOK

    pltpu.sync_copy(idx_hbm.at[pl.ds(base, TOK)], idx_vmem)
    cp_g = pltpu.async_copy(cb_hbm.at[idx_vmem], xq_vmem, sem_g)
    cp_x = pltpu.async_copy(x_hbm.at[pl.ds(base, TOK)], x_vmem, sem_x)
    cp_z = pltpu.async_copy(zero_hbm, hist_vmem, sem_z)

    acc_vmem[...] = jnp.zeros((16,), jnp.float32)
    cp_z.wait()

    @pl.loop(0, TOK, step=16)
    def _hist(t):
        plsc.addupdate_scatter(hist_vmem, [idx_vmem[pl.ds(t, 16)]],
                               jnp.ones((16,), jnp.float32))

    cp_g.wait()
    cp_x.wait()

    @pl.loop(0, TOK)
    def _rows(r):
        @pl.loop(0, D, step=16)
        def _cols(cc):
            xv = x_vmem[r, pl.ds(cc, 16)]
            qv = xq_vmem[r, pl.ds(cc, 16)]
            dv = qv - xv
            st_vmem[r, pl.ds(cc, 16)] = xv + dv
            acc_vmem[...] += dv * dv

    pltpu.sync_copy(st_vmem, st_hbm.at[pl.ds(base, TOK)])
    pltpu.sync_copy(hist_vmem, hist_hbm.at[c * 16 + s])
    pltpu.sync_copy(acc_vmem, loss_hbm.at[c * 16 + s])


def _finish_body(hp_ref, lp_ref, vq_ref, cl_ref, cm_ref, perp_ref):
    hist = jnp.sum(hp_ref[...], axis=0, keepdims=True)
    p = hist / N
    ent = jnp.sum(p * jnp.log(p + 1e-10), axis=(0, 1), keepdims=True)
    perp_ref[...] = jnp.exp(-ent)
    loss = jnp.sum(lp_ref[...], axis=(0, 1), keepdims=True)
    mse = loss / (N * D)
    cl_ref[...] = mse
    cm_ref[...] = mse
    vq_ref[...] = mse + mse * BETA


def kernel(x, codebook):
    b, d, h, w = x.shape
    xt = jnp.transpose(x, (0, 2, 3, 1))
    x_flat = xt.reshape(-1, d)
    x2 = jnp.sum(x_flat ** 2, axis=1, keepdims=True)
    c2 = jnp.sum(codebook ** 2, axis=1).reshape(1, K)
    cbt_bf = codebook.astype(jnp.bfloat16).T

    one_hot, idx2d = pl.pallas_call(
        _argmin_onehot_body,
        grid=(N // T,),
        in_specs=[
            pl.BlockSpec((T, D), lambda i: (i, 0)),
            pl.BlockSpec((T, 1), lambda i: (i, 0)),
            pl.BlockSpec((1, K), lambda i: (0, 0)),
            pl.BlockSpec((D, K), lambda i: (0, 0)),
        ],
        out_specs=[
            pl.BlockSpec((T, K), lambda i: (i, 0)),
            pl.BlockSpec((T, 1), lambda i: (i, 0)),
        ],
        out_shape=[
            jax.ShapeDtypeStruct((N, K), jnp.float32),
            jax.ShapeDtypeStruct((N, 1), jnp.int32),
        ],
        compiler_params=pltpu.CompilerParams(
            dimension_semantics=("parallel",)),
    )(x_flat, x2, c2, cbt_bf)

    idx = idx2d.reshape(N)

    sc_kernel = pl.kernel(
        _sc_lookup_kernel,
        out_type=[
            jax.ShapeDtypeStruct((N, D), jnp.float32),
            jax.ShapeDtypeStruct((NSUB, K), jnp.float32),
            jax.ShapeDtypeStruct((NSUB, 16), jnp.float32),
        ],
        mesh=plsc.VectorSubcoreMesh(core_axis_name="c", subcore_axis_name="s"),
        compiler_params=_sc_compiler_params(),
        scratch_types=[
            pltpu.VMEM((TOK,), jnp.int32),
            pltpu.VMEM((TOK, D), jnp.float32),
            pltpu.VMEM((TOK, 128), jnp.float32),
            pltpu.VMEM((TOK, D), jnp.float32),
            pltpu.VMEM((K,), jnp.float32),
            pltpu.VMEM((16,), jnp.float32),
            pltpu.SemaphoreType.DMA,
            pltpu.SemaphoreType.DMA,
            pltpu.SemaphoreType.DMA,
        ],
    )
    cb_pad = jnp.pad(codebook, ((0, 0), (0, 128 - D)))
    zero_k = jnp.zeros((K,), jnp.float32)
    st_flat, hist_parts, loss_parts = sc_kernel(idx, x_flat, cb_pad, zero_k)

    vq, cl, cm, perp = pl.pallas_call(
        _finish_body,
        out_shape=[jax.ShapeDtypeStruct((1, 1), jnp.float32)] * 4,
    )(hist_parts, loss_parts)

    x_q_st = jnp.transpose(st_flat.reshape(b, h, w, d), (0, 3, 1, 2))
    return (vq.reshape(()), cl.reshape(()), cm.reshape(()),
            x_q_st, perp.reshape(()), one_hot)


# speedup vs baseline: 2.7185x; 1.0100x over previous
import dataclasses

import jax
import jax.numpy as jnp
from jax.experimental import pallas as pl
from jax.experimental.pallas import tpu as pltpu
from jax.experimental.pallas import tpu_sc as plsc

K = 8192
D = 32
N = 9216
T = 288
NSUB = 32
TOK = N // NSUB
BETA = 0.25


def _sc_compiler_params():
    cp = pltpu.CompilerParams()
    if "needs_layout_passes" in pltpu.CompilerParams.__dataclass_fields__:
        cp = dataclasses.replace(cp, needs_layout_passes=False)
    return cp


def _argmin_onehot_body(x_ref, x2_ref, c2_ref, cbt_ref, oh_ref, idx_ref):
    xf = x_ref[...]
    xb2 = (2.0 * xf).astype(jnp.bfloat16)
    cross2 = jax.lax.dot_general(
        xb2, cbt_ref[...], (((1,), (0,)), ((), ())),
        preferred_element_type=jnp.float32)
    dist = (x2_ref[...] + c2_ref[...]) - cross2

    m = jnp.min(dist, axis=1, keepdims=True)
    iota = jax.lax.broadcasted_iota(jnp.int32, (T, K), 1)
    idxv = jnp.min(jnp.where(dist == m, iota.astype(jnp.float32), float(K)),
                   axis=1, keepdims=True)
    idxi = idxv.astype(jnp.int32)
    oh_ref[...] = (iota == idxi).astype(jnp.float32)
    idx_ref[...] = idxi


def _sc_lookup_kernel(idx_hbm, cb_hbm, zero_hbm,
                      xq_hbm, hist_hbm,
                      idx_vmem, xq_vmem, hist_vmem,
                      sem_g, sem_z):
    c = jax.lax.axis_index("c")
    s = jax.lax.axis_index("s")
    base = (c * 16 + s) * TOK

    pltpu.sync_copy(idx_hbm.at[pl.ds(base, TOK)], idx_vmem)
    cp_g = pltpu.async_copy(cb_hbm.at[idx_vmem], xq_vmem, sem_g)
    cp_z = pltpu.async_copy(zero_hbm, hist_vmem, sem_z)
    cp_z.wait()

    @pl.loop(0, TOK, step=16)
    def _hist(t):
        plsc.addupdate_scatter(hist_vmem, [idx_vmem[pl.ds(t, 16)]],
                               jnp.ones((16,), jnp.float32))

    cp_g.wait()
    pltpu.sync_copy(xq_vmem, xq_hbm.at[pl.ds(base, TOK)])
    pltpu.sync_copy(hist_vmem, hist_hbm.at[c * 16 + s])


def _finish_body(hp_ref, x_ref, xq_ref, st_ref, vq_ref, cl_ref, cm_ref,
                 perp_ref):
    hist = jnp.sum(hp_ref[...], axis=0, keepdims=True)
    p = hist / N
    ent = jnp.sum(p * jnp.log(p + 1e-10), axis=(0, 1), keepdims=True)
    perp_ref[...] = jnp.exp(-ent)
    xf = x_ref[...]
    dv = xq_ref[...][:, :D] - xf
    st_ref[...] = xf + dv
    loss = jnp.sum(dv * dv, axis=(0, 1), keepdims=True)
    mse = loss / (N * D)
    cl_ref[...] = mse
    cm_ref[...] = mse
    vq_ref[...] = mse + mse * BETA


def kernel(x, codebook):
    b, d, h, w = x.shape
    xt = jnp.transpose(x, (0, 2, 3, 1))
    x_flat = xt.reshape(-1, d)
    x2 = jnp.sum(x_flat ** 2, axis=1, keepdims=True)
    c2 = jnp.sum(codebook ** 2, axis=1).reshape(1, K)
    cbt_bf = codebook.astype(jnp.bfloat16).T

    one_hot, idx2d = pl.pallas_call(
        _argmin_onehot_body,
        grid=(N // T,),
        in_specs=[
            pl.BlockSpec((T, D), lambda i: (i, 0)),
            pl.BlockSpec((T, 1), lambda i: (i, 0)),
            pl.BlockSpec((1, K), lambda i: (0, 0)),
            pl.BlockSpec((D, K), lambda i: (0, 0)),
        ],
        out_specs=[
            pl.BlockSpec((T, K), lambda i: (i, 0)),
            pl.BlockSpec((T, 1), lambda i: (i, 0)),
        ],
        out_shape=[
            jax.ShapeDtypeStruct((N, K), jnp.float32),
            jax.ShapeDtypeStruct((N, 1), jnp.int32),
        ],
        compiler_params=pltpu.CompilerParams(
            dimension_semantics=("parallel",)),
    )(x_flat, x2, c2, cbt_bf)

    idx = idx2d.reshape(N)

    sc_kernel = pl.kernel(
        _sc_lookup_kernel,
        out_type=[
            jax.ShapeDtypeStruct((N, 128), jnp.float32),
            jax.ShapeDtypeStruct((NSUB, K), jnp.float32),
        ],
        mesh=plsc.VectorSubcoreMesh(core_axis_name="c", subcore_axis_name="s"),
        compiler_params=_sc_compiler_params(),
        scratch_types=[
            pltpu.VMEM((TOK,), jnp.int32),
            pltpu.VMEM((TOK, 128), jnp.float32),
            pltpu.VMEM((K,), jnp.float32),
            pltpu.SemaphoreType.DMA,
            pltpu.SemaphoreType.DMA,
        ],
    )
    cb_pad = jnp.pad(codebook, ((0, 0), (0, 128 - D)))
    zero_k = jnp.zeros((K,), jnp.float32)
    xq_pad, hist_parts = sc_kernel(idx, cb_pad, zero_k)

    st_flat, vq, cl, cm, perp = pl.pallas_call(
        _finish_body,
        out_shape=[jax.ShapeDtypeStruct((N, D), jnp.float32)]
        + [jax.ShapeDtypeStruct((1, 1), jnp.float32)] * 4,
    )(hist_parts, x_flat, xq_pad)

    x_q_st = jnp.transpose(st_flat.reshape(b, h, w, d), (0, 3, 1, 2))
    return (vq.reshape(()), cl.reshape(()), cm.reshape(()),
            x_q_st, perp.reshape(()), one_hot)


# speedup vs baseline: 2.8055x; 1.0320x over previous
import dataclasses

import jax
import jax.numpy as jnp
from jax.experimental import pallas as pl
from jax.experimental.pallas import tpu as pltpu
from jax.experimental.pallas import tpu_sc as plsc

K = 8192
D = 32
N = 9216
T = 288
NSUB = 32
TOK = N // NSUB
BETA = 0.25


def _sc_compiler_params():
    cp = pltpu.CompilerParams()
    if "needs_layout_passes" in pltpu.CompilerParams.__dataclass_fields__:
        cp = dataclasses.replace(cp, needs_layout_passes=False)
    return cp


def _argmin_onehot_body(x_ref, c2_ref, cbt_ref, oh_ref, idx_ref):
    xf = x_ref[...]
    xb2 = (2.0 * xf).astype(jnp.bfloat16)
    cross2 = jax.lax.dot_general(
        xb2, cbt_ref[...], (((1,), (0,)), ((), ())),
        preferred_element_type=jnp.float32)
    x2 = jnp.sum(xf ** 2, axis=1, keepdims=True)
    dist = (x2 + c2_ref[...]) - cross2

    m = jnp.min(dist, axis=1, keepdims=True)
    iota = jax.lax.broadcasted_iota(jnp.int32, (T, K), 1)
    idxv = jnp.min(jnp.where(dist == m, iota.astype(jnp.float32), float(K)),
                   axis=1, keepdims=True)
    idxi = idxv.astype(jnp.int32)
    oh_ref[...] = (iota == idxi).astype(jnp.float32)
    idx_ref[...] = idxi


def _sc_lookup_kernel(idx_hbm, cb_hbm, zero_hbm,
                      xq_hbm, hist_hbm,
                      idx_vmem, xq_vmem, hist_vmem,
                      sem_g, sem_z):
    c = jax.lax.axis_index("c")
    s = jax.lax.axis_index("s")
    base = (c * 16 + s) * TOK

    pltpu.sync_copy(idx_hbm.at[pl.ds(base, TOK)], idx_vmem)
    cp_g = pltpu.async_copy(cb_hbm.at[idx_vmem], xq_vmem, sem_g)
    cp_z = pltpu.async_copy(zero_hbm, hist_vmem, sem_z)
    cp_z.wait()

    @pl.loop(0, TOK, step=16)
    def _hist(t):
        plsc.addupdate_scatter(hist_vmem, [idx_vmem[pl.ds(t, 16)]],
                               jnp.ones((16,), jnp.float32))

    cp_g.wait()
    pltpu.sync_copy(xq_vmem, xq_hbm.at[pl.ds(base, TOK)])
    pltpu.sync_copy(hist_vmem, hist_hbm.at[c * 16 + s])


def _finish_body(hp_ref, x_ref, xq_ref, st_ref, vq_ref, cl_ref, cm_ref,
                 perp_ref):
    hist = jnp.sum(hp_ref[...], axis=0, keepdims=True)
    p = hist / N
    ent = jnp.sum(p * jnp.log(p + 1e-10), axis=(0, 1), keepdims=True)
    perp_ref[...] = jnp.exp(-ent)
    xf = x_ref[...]
    dv = xq_ref[...][:, :D] - xf
    st_ref[...] = xf + dv
    loss = jnp.sum(dv * dv, axis=(0, 1), keepdims=True)
    mse = loss / (N * D)
    cl_ref[...] = mse
    cm_ref[...] = mse
    vq_ref[...] = mse + mse * BETA


def kernel(x, codebook):
    b, d, h, w = x.shape
    xt = jnp.transpose(x, (0, 2, 3, 1))
    x_flat = xt.reshape(-1, d)
    c2 = jnp.sum(codebook ** 2, axis=1).reshape(1, K)
    cbt_bf = codebook.astype(jnp.bfloat16).T

    one_hot, idx2d = pl.pallas_call(
        _argmin_onehot_body,
        grid=(N // T,),
        in_specs=[
            pl.BlockSpec((T, D), lambda i: (i, 0)),
            pl.BlockSpec((1, K), lambda i: (0, 0)),
            pl.BlockSpec((D, K), lambda i: (0, 0)),
        ],
        out_specs=[
            pl.BlockSpec((T, K), lambda i: (i, 0)),
            pl.BlockSpec((T, 1), lambda i: (i, 0)),
        ],
        out_shape=[
            jax.ShapeDtypeStruct((N, K), jnp.float32),
            jax.ShapeDtypeStruct((N, 1), jnp.int32),
        ],
        compiler_params=pltpu.CompilerParams(
            dimension_semantics=("parallel",)),
    )(x_flat, c2, cbt_bf)

    idx = idx2d.reshape(N)

    sc_kernel = pl.kernel(
        _sc_lookup_kernel,
        out_type=[
            jax.ShapeDtypeStruct((N, 128), jnp.float32),
            jax.ShapeDtypeStruct((NSUB, K), jnp.float32),
        ],
        mesh=plsc.VectorSubcoreMesh(core_axis_name="c", subcore_axis_name="s"),
        compiler_params=_sc_compiler_params(),
        scratch_types=[
            pltpu.VMEM((TOK,), jnp.int32),
            pltpu.VMEM((TOK, 128), jnp.float32),
            pltpu.VMEM((K,), jnp.float32),
            pltpu.SemaphoreType.DMA,
            pltpu.SemaphoreType.DMA,
        ],
    )
    cb_pad = jnp.pad(codebook, ((0, 0), (0, 128 - D)))
    zero_k = jnp.zeros((K,), jnp.float32)
    xq_pad, hist_parts = sc_kernel(idx, cb_pad, zero_k)

    st_flat, vq, cl, cm, perp = pl.pallas_call(
        _finish_body,
        out_shape=[jax.ShapeDtypeStruct((N, D), jnp.float32)]
        + [jax.ShapeDtypeStruct((1, 1), jnp.float32)] * 4,
    )(hist_parts, x_flat, xq_pad)

    x_q_st = jnp.transpose(st_flat.reshape(b, h, w, d), (0, 3, 1, 2))
    return (vq.reshape(()), cl.reshape(()), cm.reshape(()),
            x_q_st, perp.reshape(()), one_hot)
